# Initial kernel scaffold; baseline (speedup 1.0000x reference)
import jax
import jax.numpy as jnp
from jax.experimental import pallas as pl


def kernel(points, label_shape):
    raise NotImplementedError("write your pallas kernel here")


# speedup vs baseline: 11.0093x; 11.0093x over previous
import functools

import jax
import jax.numpy as jnp
import numpy as np
from jax import lax
from jax.experimental import pallas as pl
from jax.experimental.pallas import tpu as pltpu
from jax.experimental.pallas import tpu_sc as plsc

_B = 4
_N = 12800
_R, _C = 100, 128
_GEOM = [-40.0, 40.0, -40.0, 40.0, -2.0, 2.0]
_MEAN = np.array([-0.25, 0.01, 0.26, 0.46, 0.26, 0.62, 0.69, 1.46], dtype=np.float32)
_STD = np.array([0.93, 0.26, 0.53, 0.89, 1.11, 0.13, 0.16, 0.19], dtype=np.float32)


def _eul2rotm(rx, ry, rz):
    cx, sx = jnp.cos(rx), jnp.sin(rx)
    cy, sy = jnp.cos(ry), jnp.sin(ry)
    cz, sz = jnp.cos(rz), jnp.sin(rz)
    r00 = cy * cz
    r01 = sy * sx * cz - sz * cx
    r02 = sy * cx * cz + sz * sx
    r10 = cy * sz
    r11 = sy * sx * sz + cz * cx
    r12 = sy * cx * sz - cz * sx
    r20 = -sy
    r21 = cy * sx
    r22 = cy * cx
    row0 = jnp.stack([r00, r01, r02], axis=-1)
    row1 = jnp.stack([r10, r11, r12], axis=-1)
    row2 = jnp.stack([r20, r21, r22], axis=-1)
    return jnp.stack([row0, row1, row2], axis=-2)


def _boxes_to_corners(boxes):
    xc, yc, zc, lx, ly, lz, rx, ry, rz = [boxes[..., i] for i in range(9)]
    sgn_x = jnp.array([1., 1., -1., -1., 1., 1., -1., -1.], dtype=jnp.float32) * 0.5
    sgn_y = jnp.array([1., 1., 1., 1., -1., -1., -1., -1.], dtype=jnp.float32) * 0.5
    sgn_z = jnp.array([1., -1., 1., -1., 1., -1., 1., -1.], dtype=jnp.float32) * 0.5
    xs = lx[..., None] * sgn_x
    ys = ly[..., None] * sgn_y
    zs = lz[..., None] * sgn_z
    xyz = jnp.stack([xs, ys, zs], axis=-1)
    R = _eul2rotm(rx, ry, rz)
    rotated = jnp.einsum('bnij,bnkj->bnki', R, xyz)
    centers = jnp.stack([xc, yc, zc], axis=-1)[..., None, :]
    return rotated + centers


def _decode5(points):
    Bc, C, Z, Y, X = points.shape
    xg = (_GEOM[1] - _GEOM[0]) / float(X)
    yg = (_GEOM[3] - _GEOM[2]) / float(Y)
    zg = (_GEOM[5] - _GEOM[4]) / float(Z)
    mean = jnp.asarray(_MEAN).reshape(1, 8, 1, 1, 1)
    std = jnp.asarray(_STD).reshape(1, 8, 1, 1, 1)
    cls_ = points[:, 0:1]
    reg = points[:, 1:9] * std + mean
    cos_t, sin_t, dx, dy, dz, log_w, log_h, log_l = [reg[:, i:i + 1] for i in range(8)]
    theta = jnp.arctan2(sin_t, cos_t)
    x = jnp.arange(_GEOM[0], _GEOM[1], xg, dtype=jnp.float32)
    y = jnp.arange(_GEOM[2], _GEOM[3], yg, dtype=jnp.float32)
    z = jnp.arange(_GEOM[4], _GEOM[5], zg, dtype=jnp.float32)
    zz, yy, xx = jnp.meshgrid(z, y, x, indexing='ij')
    centre_z = zz + dz
    centre_y = yy + dy
    centre_x = xx + dx
    l = jnp.exp(log_l)
    h = jnp.exp(log_h)
    w = jnp.exp(log_w)
    return cls_, theta, centre_x, centre_y, centre_z, w, h, l


def _batch_boxes(dec, b, flat_idx):
    cls_, theta, cx, cy, cz, w, h, l = dec

    def g(t):
        return t[b].reshape(-1)[flat_idx]

    sf = g(cls_)
    tf = g(theta)
    zeros = jnp.zeros_like(tf)
    box_theta = jnp.stack([g(cx), g(cy), g(cz), g(w), g(h), g(l), zeros, tf, zeros],
                          axis=1)[None]
    corners = _boxes_to_corners(box_theta)
    xmin = corners[..., 0].min(axis=-1)
    ymin = corners[..., 1].min(axis=-1)
    zmin = corners[..., 2].min(axis=-1)
    xmax = corners[..., 0].max(axis=-1)
    ymax = corners[..., 1].max(axis=-1)
    zmax = corners[..., 2].max(axis=-1)
    xy_box = jnp.stack([xmin, ymin, xmax, ymax], axis=-1).reshape(-1, 4)
    yz_box = jnp.stack([ymin, zmin, ymax, zmax], axis=-1).reshape(-1, 4)
    return box_theta, sf, xy_box, yz_box


def _nms_while(x1, y1, x2, y2, area, sf, valid, flat):
    def cond(st):
        alive_i, _ = st
        return jnp.max(alive_i) > 0

    def body(st):
        alive_i, keep_i = st
        alive = alive_i != 0
        s = jnp.where(alive, sf, -jnp.inf)
        m = jnp.max(s)
        idx = jnp.min(jnp.where(s == m, flat, jnp.int32(_N)))
        mi = flat == idx
        neg = jnp.float32(-jnp.inf)
        x1i = jnp.max(jnp.where(mi, x1, neg))
        y1i = jnp.max(jnp.where(mi, y1, neg))
        x2i = jnp.max(jnp.where(mi, x2, neg))
        y2i = jnp.max(jnp.where(mi, y2, neg))
        ai = jnp.max(jnp.where(mi, area, neg))
        xx1 = jnp.maximum(x1i, x1)
        yy1 = jnp.maximum(y1i, y1)
        xx2 = jnp.minimum(x2i, x2)
        yy2 = jnp.minimum(y2i, y2)
        inter = jnp.maximum(0.0, xx2 - xx1) * jnp.maximum(0.0, yy2 - yy1)
        iou = inter / (ai + area - inter)
        alive = alive & (iou <= 0.5) & jnp.logical_not(mi)
        keep_i = keep_i | mi.astype(jnp.int32)
        return alive.astype(jnp.int32), keep_i

    valid_i = valid.astype(jnp.int32)
    keep0 = jnp.zeros_like(valid_i)
    _, keep_i = lax.while_loop(cond, body, (valid_i, keep0))
    return keep_i != 0


def _tc_nms_body(boxes_ref, keep_ref):
    f = [boxes_ref[0, k] for k in range(9)]
    x1a, y1a, x2a, y2a, x1b, y1b, x2b, y2b, sf = f
    area_a = (x2a - x1a) * (y2a - y1a)
    area_b = (x2b - x1b) * (y2b - y1b)
    valid = sf > 0.5
    flat = (lax.broadcasted_iota(jnp.int32, (_R, _C), 0) * _C
            + lax.broadcasted_iota(jnp.int32, (_R, _C), 1))
    kxy = _nms_while(x1a, y1a, x2a, y2a, area_a, sf, valid, flat)
    kyz = _nms_while(x1b, y1b, x2b, y2b, area_b, sf, valid, flat)
    keep_ref[0] = (kxy | kyz).astype(jnp.int32)


_tc_nms = pl.pallas_call(
    _tc_nms_body,
    grid=(_B,),
    in_specs=[pl.BlockSpec((1, 9, _R, _C), lambda b: (b, 0, 0, 0))],
    out_specs=pl.BlockSpec((1, _R, _C), lambda b: (b, 0, 0)),
    out_shape=jax.ShapeDtypeStruct((_B, _R, _C), jnp.int32),
)

_NT = _B * 10
_NW = 32


def _sc_compact_body(cols_hbm, keep_hbm, out_hbm, vals_v, kp_v, out_v):
    wid = lax.axis_index("s") * 2 + lax.axis_index("c")
    for t0 in range(2):
        task = wid + t0 * _NW

        @pl.when(task < _NT)
        def _():
            batch = task // 10
            pltpu.sync_copy(cols_hbm.at[task], vals_v)
            pltpu.sync_copy(keep_hbm.at[batch], kp_v)

            def zbody(i, carry):
                out_v[pl.ds(i * 16, 16)] = jnp.zeros((16,), jnp.float32)
                return carry

            lax.fori_loop(0, (_N + 16) // 16, zbody, 0)

            lane = lax.iota(jnp.int32, 16)

            def cbody(i, cnt):
                vals = vals_v[pl.ds(i * 16, 16)]
                kp = kp_v[pl.ds(i * 16, 16)] > 0
                prefix = plsc.cumsum(kp.astype(jnp.int32))
                dest = jnp.where(kp, cnt + prefix - 1, _N + lane)
                plsc.store_scatter(out_v, [dest], vals)
                return cnt + jnp.sum(kp.astype(jnp.int32))

            lax.fori_loop(0, _N // 16, cbody, jnp.int32(0))
            pltpu.sync_copy(out_v.at[pl.ds(0, _N)], out_hbm.at[task])


@functools.cache
def _sc_compact():
    return functools.partial(
        pl.kernel,
        mesh=plsc.VectorSubcoreMesh(core_axis_name="c", subcore_axis_name="s"),
        compiler_params=pltpu.CompilerParams(needs_layout_passes=False),
        out_type=jax.ShapeDtypeStruct((_NT, _N), jnp.float32),
        scratch_types=[
            pltpu.VMEM((_N,), jnp.float32),
            pltpu.VMEM((_N,), jnp.int32),
            pltpu.VMEM((_N + 16,), jnp.float32),
        ],
    )(_sc_compact_body)


def kernel(points, label_shape):
    dec = _decode5(points)
    flat_idx = jnp.arange(_N, dtype=jnp.int32) + label_shape[0] * 0
    nms_rows, col_rows = [], []
    for b in range(_B):
        box_theta, sf, xy_box, yz_box = _batch_boxes(dec, b, flat_idx)
        nms_rows.append(jnp.concatenate(
            [xy_box.T, yz_box.T, sf[None]], axis=0))
        col_rows.append(jnp.concatenate(
            [jnp.transpose(box_theta[0], (1, 0)), sf[None]], axis=0))
    nms_in = jnp.stack(nms_rows, axis=0)
    cols = jnp.stack(col_rows, axis=0).reshape(_NT, _N)

    keep = _tc_nms(nms_in.reshape(_B, 9, _R, _C)).reshape(_B, _N)
    out = _sc_compact()(cols, keep).reshape(_B, 10, _N)
    bb = jnp.transpose(out[:, :9], (0, 2, 1))
    ss = out[:, 9]
    return bb, ss


# speedup vs baseline: 15.6815x; 1.4244x over previous
import functools

import jax
import jax.numpy as jnp
import numpy as np
from jax import lax
from jax.experimental import pallas as pl
from jax.experimental.pallas import tpu as pltpu
from jax.experimental.pallas import tpu_sc as plsc

_B = 4
_N = 12800
_R, _C = 100, 128
_GEOM = [-40.0, 40.0, -40.0, 40.0, -2.0, 2.0]
_MEAN = np.array([-0.25, 0.01, 0.26, 0.46, 0.26, 0.62, 0.69, 1.46], dtype=np.float32)
_STD = np.array([0.93, 0.26, 0.53, 0.89, 1.11, 0.13, 0.16, 0.19], dtype=np.float32)


def _eul2rotm(rx, ry, rz):
    cx, sx = jnp.cos(rx), jnp.sin(rx)
    cy, sy = jnp.cos(ry), jnp.sin(ry)
    cz, sz = jnp.cos(rz), jnp.sin(rz)
    r00 = cy * cz
    r01 = sy * sx * cz - sz * cx
    r02 = sy * cx * cz + sz * sx
    r10 = cy * sz
    r11 = sy * sx * sz + cz * cx
    r12 = sy * cx * sz - cz * sx
    r20 = -sy
    r21 = cy * sx
    r22 = cy * cx
    row0 = jnp.stack([r00, r01, r02], axis=-1)
    row1 = jnp.stack([r10, r11, r12], axis=-1)
    row2 = jnp.stack([r20, r21, r22], axis=-1)
    return jnp.stack([row0, row1, row2], axis=-2)


def _boxes_to_corners(boxes):
    xc, yc, zc, lx, ly, lz, rx, ry, rz = [boxes[..., i] for i in range(9)]
    sgn_x = jnp.array([1., 1., -1., -1., 1., 1., -1., -1.], dtype=jnp.float32) * 0.5
    sgn_y = jnp.array([1., 1., 1., 1., -1., -1., -1., -1.], dtype=jnp.float32) * 0.5
    sgn_z = jnp.array([1., -1., 1., -1., 1., -1., 1., -1.], dtype=jnp.float32) * 0.5
    xs = lx[..., None] * sgn_x
    ys = ly[..., None] * sgn_y
    zs = lz[..., None] * sgn_z
    xyz = jnp.stack([xs, ys, zs], axis=-1)
    R = _eul2rotm(rx, ry, rz)
    rotated = jnp.einsum('bnij,bnkj->bnki', R, xyz)
    centers = jnp.stack([xc, yc, zc], axis=-1)[..., None, :]
    return rotated + centers


def _decode5(points):
    Bc, C, Z, Y, X = points.shape
    xg = (_GEOM[1] - _GEOM[0]) / float(X)
    yg = (_GEOM[3] - _GEOM[2]) / float(Y)
    zg = (_GEOM[5] - _GEOM[4]) / float(Z)
    mean = jnp.asarray(_MEAN).reshape(1, 8, 1, 1, 1)
    std = jnp.asarray(_STD).reshape(1, 8, 1, 1, 1)
    cls_ = points[:, 0:1]
    reg = points[:, 1:9] * std + mean
    cos_t, sin_t, dx, dy, dz, log_w, log_h, log_l = [reg[:, i:i + 1] for i in range(8)]
    theta = jnp.arctan2(sin_t, cos_t)
    x = jnp.arange(_GEOM[0], _GEOM[1], xg, dtype=jnp.float32)
    y = jnp.arange(_GEOM[2], _GEOM[3], yg, dtype=jnp.float32)
    z = jnp.arange(_GEOM[4], _GEOM[5], zg, dtype=jnp.float32)
    zz, yy, xx = jnp.meshgrid(z, y, x, indexing='ij')
    centre_z = zz + dz
    centre_y = yy + dy
    centre_x = xx + dx
    l = jnp.exp(log_l)
    h = jnp.exp(log_h)
    w = jnp.exp(log_w)
    return cls_, theta, centre_x, centre_y, centre_z, w, h, l


def _batch_boxes(dec, b, flat_idx):
    cls_, theta, cx, cy, cz, w, h, l = dec

    def g(t):
        return t[b].reshape(-1)[flat_idx]

    sf = g(cls_)
    tf = g(theta)
    zeros = jnp.zeros_like(tf)
    box_theta = jnp.stack([g(cx), g(cy), g(cz), g(w), g(h), g(l), zeros, tf, zeros],
                          axis=1)[None]
    corners = _boxes_to_corners(box_theta)
    xmin = corners[..., 0].min(axis=-1)
    ymin = corners[..., 1].min(axis=-1)
    zmin = corners[..., 2].min(axis=-1)
    xmax = corners[..., 0].max(axis=-1)
    ymax = corners[..., 1].max(axis=-1)
    zmax = corners[..., 2].max(axis=-1)
    xy_box = jnp.stack([xmin, ymin, xmax, ymax], axis=-1).reshape(-1, 4)
    yz_box = jnp.stack([ymin, zmin, ymax, zmax], axis=-1).reshape(-1, 4)
    return box_theta, sf, xy_box, yz_box


def _nms_while(boxes_ref, k0, x1, y1, x2, y2, area, sf, valid, flat):
    neg = jnp.float32(-jnp.inf)

    def argmax(s):
        m = jnp.max(s)
        idx = jnp.min(jnp.where(s == m, flat, jnp.int32(_N)))
        return m, idx

    def cond(st):
        _, _, m, _ = st
        return m > neg

    def body(st):
        s, keep_i, m, idx = st
        r = idx // _C
        c = idx % _C
        neg_row = jnp.full((1, _C), neg, jnp.float32)
        cmask = lax.broadcasted_iota(jnp.int32, (1, _C), 1) == c

        def fetch(k):
            row = boxes_ref[0, k, pl.ds(r, 1), :]
            return jnp.max(jnp.where(cmask, row, neg_row))

        x1i = fetch(k0)
        y1i = fetch(k0 + 1)
        x2i = fetch(k0 + 2)
        y2i = fetch(k0 + 3)
        ai = (x2i - x1i) * (y2i - y1i)
        mi = flat == idx
        xx1 = jnp.maximum(x1i, x1)
        yy1 = jnp.maximum(y1i, y1)
        xx2 = jnp.minimum(x2i, x2)
        yy2 = jnp.minimum(y2i, y2)
        inter = jnp.maximum(0.0, xx2 - xx1) * jnp.maximum(0.0, yy2 - yy1)
        iou = inter / (ai + area - inter)
        s = jnp.where((iou <= 0.5) & jnp.logical_not(mi), s, neg)
        keep_i = keep_i | mi.astype(jnp.int32)
        m, idx = argmax(s)
        return s, keep_i, m, idx

    s0 = jnp.where(valid, sf, neg)
    keep0 = jnp.zeros((_R, _C), jnp.int32)
    m0, idx0 = argmax(s0)
    _, keep_i, _, _ = lax.while_loop(cond, body, (s0, keep0, m0, idx0))
    return keep_i


def _tc_nms_body(boxes_ref, keep_ref):
    f = [boxes_ref[0, k] for k in range(9)]
    x1a, y1a, x2a, y2a, x1b, y1b, x2b, y2b, sf = f
    area_a = (x2a - x1a) * (y2a - y1a)
    area_b = (x2b - x1b) * (y2b - y1b)
    valid = sf > 0.5
    flat = (lax.broadcasted_iota(jnp.int32, (_R, _C), 0) * _C
            + lax.broadcasted_iota(jnp.int32, (_R, _C), 1))
    kxy = _nms_while(boxes_ref, 0, x1a, y1a, x2a, y2a, area_a, sf, valid, flat)
    kyz = _nms_while(boxes_ref, 4, x1b, y1b, x2b, y2b, area_b, sf, valid, flat)
    keep_ref[0] = kxy | kyz


_tc_nms = pl.pallas_call(
    _tc_nms_body,
    grid=(_B,),
    in_specs=[pl.BlockSpec((1, 9, _R, _C), lambda b: (b, 0, 0, 0))],
    out_specs=pl.BlockSpec((1, _R, _C), lambda b: (b, 0, 0)),
    out_shape=jax.ShapeDtypeStruct((_B, _R, _C), jnp.int32),
)

_NT = _B * 10
_NW = 32


def _sc_compact_body(cols_hbm, keep_hbm, out_hbm, vals_v, kp_v, out_v):
    wid = lax.axis_index("s") * 2 + lax.axis_index("c")
    for t0 in range(2):
        task = wid + t0 * _NW

        @pl.when(task < _NT)
        def _():
            batch = task // 10
            pltpu.sync_copy(cols_hbm.at[task], vals_v)
            pltpu.sync_copy(keep_hbm.at[batch], kp_v)

            def zbody(i, carry):
                out_v[pl.ds(i * 16, 16)] = jnp.zeros((16,), jnp.float32)
                return carry

            lax.fori_loop(0, (_N + 16) // 16, zbody, 0)

            lane = lax.iota(jnp.int32, 16)

            def cbody(i, cnt):
                vals = vals_v[pl.ds(i * 16, 16)]
                kp = kp_v[pl.ds(i * 16, 16)] > 0
                prefix = plsc.cumsum(kp.astype(jnp.int32))
                dest = jnp.where(kp, cnt + prefix - 1, _N + lane)
                plsc.store_scatter(out_v, [dest], vals)
                return cnt + jnp.sum(kp.astype(jnp.int32))

            lax.fori_loop(0, _N // 16, cbody, jnp.int32(0))
            pltpu.sync_copy(out_v.at[pl.ds(0, _N)], out_hbm.at[task])


@functools.cache
def _sc_compact():
    return functools.partial(
        pl.kernel,
        mesh=plsc.VectorSubcoreMesh(core_axis_name="c", subcore_axis_name="s"),
        compiler_params=pltpu.CompilerParams(needs_layout_passes=False),
        out_type=jax.ShapeDtypeStruct((_NT, _N), jnp.float32),
        scratch_types=[
            pltpu.VMEM((_N,), jnp.float32),
            pltpu.VMEM((_N,), jnp.int32),
            pltpu.VMEM((_N + 16,), jnp.float32),
        ],
    )(_sc_compact_body)


def kernel(points, label_shape):
    dec = _decode5(points)
    flat_idx = jnp.arange(_N, dtype=jnp.int32) + label_shape[0] * 0
    nms_rows, col_rows = [], []
    for b in range(_B):
        box_theta, sf, xy_box, yz_box = _batch_boxes(dec, b, flat_idx)
        nms_rows.append(jnp.concatenate(
            [xy_box.T, yz_box.T, sf[None]], axis=0))
        col_rows.append(jnp.concatenate(
            [jnp.transpose(box_theta[0], (1, 0)), sf[None]], axis=0))
    nms_in = jnp.stack(nms_rows, axis=0)
    cols = jnp.stack(col_rows, axis=0).reshape(_NT, _N)

    keep = _tc_nms(nms_in.reshape(_B, 9, _R, _C)).reshape(_B, _N)
    out = _sc_compact()(cols, keep).reshape(_B, 10, _N)
    bb = jnp.transpose(out[:, :9], (0, 2, 1))
    ss = out[:, 9]
    return bb, ss
